# baseline (speedup 1.0000x reference)
import functools

import jax
import jax.numpy as jnp
from jax import lax
from jax.experimental import pallas as pl
from jax.experimental.pallas import tpu as pltpu
from jax.experimental.pallas import tpu_sc as plsc

f32 = jnp.float32
i32 = jnp.int32

_N = 10000
_E = 320000
_IN = 128
_HID = 256
_OUT = 128

_NP = 10240
_RB = 1024
_RPT = _NP // 16
_K = 128
_EP = 323584
_NSC = 2
_F = 128

_mesh = plsc.VectorSubcoreMesh(core_axis_name="c", subcore_axis_name="s")


@functools.partial(
    pl.kernel,
    out_type=jax.ShapeDtypeStruct((_NSC * _NP, _F), f32),
    mesh=_mesh,
    scratch_types=[
        pltpu.VMEM_SHARED((_NP, _F), f32),
        pltpu.VMEM((_K,), i32),
        pltpu.VMEM((_K,), i32),
        pltpu.VMEM((16,), i32),
        pltpu.VMEM((16,), i32),
        pltpu.VMEM((_K, _F), f32),
        pltpu.VMEM((_K, _F), f32),
        pltpu.SemaphoreType.DMA,
    ],
)
def _sc_scatter(g_hbm, src_hbm, dst_hbm, zeros_hbm, offv_hbm, modev_hbm,
                out_hbm, acc, srcv, dstv, offv, modev, rowsv, stage, sem):
    c = lax.axis_index("c")
    s = lax.axis_index("s")
    row0 = s * _RPT
    pltpu.sync_copy(zeros_hbm, acc.at[pl.ds(row0, _RPT)])
    pltpu.sync_copy(offv_hbm, offv)
    pltpu.sync_copy(modev_hbm, modev)
    plsc.subcore_barrier()

    per_tile = _EP // 16
    base = s * per_tile
    cbit = c + 1
    off16 = offv[...] * c
    keep16 = (modev[...] & cbit) != 0
    trash16 = jnp.full((16,), _N, i32)

    def body(t, carry):
        off = base + t * _K
        pltpu.sync_copy(src_hbm.at[pl.ds(off, _K)], srcv)
        pltpu.sync_copy(dst_hbm.at[pl.ds(off, _K)], dstv)
        for j in range(_K // 16):
            sl = pl.ds(j * 16, 16)
            srcv[sl] = srcv[sl] + off16
            dstv[sl] = jnp.where(keep16, dstv[sl], trash16)
        pltpu.async_copy(g_hbm.at[srcv], rowsv, sem).wait()
        pltpu.sync_copy(rowsv, acc.at[dstv], add=True)
        return carry

    lax.fori_loop(0, per_tile // _K, body, 0)
    plsc.subcore_barrier()
    for t in range(_RPT // _K):
        r = row0 + t * _K
        pltpu.sync_copy(acc.at[pl.ds(r, _K)], stage)
        pltpu.sync_copy(stage, out_hbm.at[pl.ds(c * _NP + r, _K)])


def _tc1_body(x_ref, w_ref, deg_ref, gs_ref, gf_ref, dinv_ref):
    deg = deg_ref[0, :, 0:1] + deg_ref[1, :, 0:1] + 1.0
    dinv = lax.rsqrt(deg)
    h = jnp.dot(x_ref[...], w_ref[...], preferred_element_type=f32)
    g = h * dinv
    half = _HID // 2
    gs_ref[0] = g[:, :half]
    gs_ref[1] = g[:, half:]
    gf_ref[...] = g
    dinv_ref[...] = dinv


def _tc1(x_pad, W1, degacc):
    grid = (_NP // _RB,)
    return pl.pallas_call(
        _tc1_body,
        grid=grid,
        in_specs=[
            pl.BlockSpec((_RB, _IN), lambda i: (i, 0)),
            pl.BlockSpec((_IN, _HID), lambda i: (0, 0)),
            pl.BlockSpec((2, _RB, 128), lambda i: (0, i, 0)),
        ],
        out_specs=[
            pl.BlockSpec((2, _RB, _HID // 2), lambda i: (0, i, 0)),
            pl.BlockSpec((_RB, _HID), lambda i: (i, 0)),
            pl.BlockSpec((_RB, 1), lambda i: (i, 0)),
        ],
        out_shape=[
            jax.ShapeDtypeStruct((2, _NP, _HID // 2), f32),
            jax.ShapeDtypeStruct((_NP, _HID), f32),
            jax.ShapeDtypeStruct((_NP, 1), f32),
        ],
    )(x_pad, W1, degacc)


def _tc2_body(acc_ref, gf_ref, dinv_ref, b_ref, w_ref, g2f_ref):
    accf = jnp.concatenate([acc_ref[0], acc_ref[1]], axis=1)
    dinv = dinv_ref[...]
    h2 = jnp.maximum(dinv * (accf + gf_ref[...]) + b_ref[...], 0.0)
    g2f_ref[...] = jnp.dot(h2, w_ref[...], preferred_element_type=f32) * dinv


def _tc2(acc1, g1f, dinv, b1, W2):
    grid = (_NP // _RB,)
    return pl.pallas_call(
        _tc2_body,
        grid=grid,
        in_specs=[
            pl.BlockSpec((2, _RB, _HID // 2), lambda i: (0, i, 0)),
            pl.BlockSpec((_RB, _HID), lambda i: (i, 0)),
            pl.BlockSpec((_RB, 1), lambda i: (i, 0)),
            pl.BlockSpec((1, _HID), lambda i: (0, 0)),
            pl.BlockSpec((_HID, _OUT), lambda i: (0, 0)),
        ],
        out_specs=pl.BlockSpec((_RB, _OUT), lambda i: (i, 0)),
        out_shape=jax.ShapeDtypeStruct((_NSC * _NP, _OUT), f32),
    )(acc1, g1f, dinv, b1, W2)


def _tc3_body(acc_ref, g2f_ref, dinv_ref, b_ref, out_ref):
    accf = acc_ref[0] + acc_ref[1]
    out_ref[...] = dinv_ref[...] * (accf + g2f_ref[...]) + b_ref[...]


def _tc3(acc2, g2f, dinv, b2):
    grid = (_NP // _RB,)
    return pl.pallas_call(
        _tc3_body,
        grid=grid,
        in_specs=[
            pl.BlockSpec((2, _RB, _OUT), lambda i: (0, i, 0)),
            pl.BlockSpec((_RB, _OUT), lambda i: (i, 0)),
            pl.BlockSpec((_RB, 1), lambda i: (i, 0)),
            pl.BlockSpec((1, _OUT), lambda i: (0, 0)),
        ],
        out_specs=pl.BlockSpec((_RB, _OUT), lambda i: (i, 0)),
        out_shape=jax.ShapeDtypeStruct((_NP, _OUT), f32),
    )(acc2, g2f, dinv, b2)


@jax.jit
def _run(x, edge_index, W1, b1, W2, b2):
    ei = edge_index.astype(i32)
    pad = jnp.full((_EP - _E,), _N, dtype=i32)
    src = jnp.concatenate([ei[0], pad])
    dst = jnp.concatenate([ei[1], pad])

    x_pad = jnp.zeros((_NP, _IN), f32).at[:_N].set(x)

    zeros_acc = jnp.zeros((_RPT, _F), f32)
    keep_all = jnp.full((16,), 3, i32)
    keep_eo = jnp.where(jnp.arange(16) % 2 == 0, 1, 2).astype(i32)
    off_np = jnp.full((16,), _NP, i32)
    off_0 = jnp.zeros((16,), i32)

    onehot_tbl = jnp.zeros((_NSC * _NP, _F), f32).at[0, 0].set(1.0)
    src0 = jnp.zeros((_EP,), i32)
    degacc = _sc_scatter(onehot_tbl, src0, dst, zeros_acc, off_0, keep_eo)
    degacc = degacc.reshape(_NSC, _NP, _F)

    g1s, g1f, dinv = _tc1(x_pad, W1, degacc)

    acc1 = _sc_scatter(g1s.reshape(_NSC * _NP, _F), src, dst, zeros_acc,
                       off_np, keep_all)
    acc1 = acc1.reshape(_NSC, _NP, _F)

    g2pad = _tc2(acc1, g1f, dinv, b1.reshape(1, _HID), W2)

    acc2 = _sc_scatter(g2pad, src, dst, zeros_acc, off_0, keep_eo)
    acc2 = acc2.reshape(_NSC, _NP, _OUT)

    out = _tc3(acc2, g2pad, dinv, b2.reshape(1, _OUT))
    return out[:_N]


def kernel(x, edge_index, W1, b1, W2, b2):
    return _run(x, edge_index, W1, b1, W2, b2)


# speedup vs baseline: 12.6370x; 12.6370x over previous
import functools

import jax
import jax.numpy as jnp
from jax import lax
from jax.experimental import pallas as pl
from jax.experimental.pallas import tpu as pltpu
from jax.experimental.pallas import tpu_sc as plsc

f32 = jnp.float32
i32 = jnp.int32

_N = 10000
_E = 320000
_IN = 128
_HID = 256
_OUT = 128

_NP = 10240
_RB = 1024
_RPT = _NP // 16
_K = 128
_EP = 323584
_NSC = 2
_F = 128

_mesh = plsc.VectorSubcoreMesh(core_axis_name="c", subcore_axis_name="s")


@functools.partial(
    pl.kernel,
    out_type=jax.ShapeDtypeStruct((_NSC * _NP, _F), f32),
    mesh=_mesh,
    scratch_types=[
        pltpu.VMEM_SHARED((_NP, _F), f32),
        pltpu.VMEM((_K,), i32),
        pltpu.VMEM((_K,), i32),
        pltpu.VMEM((16,), i32),
        pltpu.VMEM((16,), i32),
        pltpu.VMEM((_K, _F), f32),
        pltpu.VMEM((_K, _F), f32),
        pltpu.SemaphoreType.DMA,
    ],
)
def _sc_scatter(g_hbm, src_hbm, dst_hbm, zeros_hbm, offv_hbm, modev_hbm,
                out_hbm, acc, srcv, dstv, offv, modev, rowsv, stage, sem):
    c = lax.axis_index("c")
    s = lax.axis_index("s")
    row0 = s * _RPT
    pltpu.sync_copy(zeros_hbm, acc.at[pl.ds(row0, _RPT)])
    pltpu.sync_copy(offv_hbm, offv)
    pltpu.sync_copy(modev_hbm, modev)
    plsc.subcore_barrier()

    per_tile = _EP // 16
    base = s * per_tile
    cbit = c + 1
    off16 = offv[...] * c
    keep16 = (modev[...] & cbit) != 0
    trash16 = jnp.full((16,), _N, i32)

    def body(t, carry):
        off = base + t * _K
        pltpu.sync_copy(src_hbm.at[pl.ds(off, _K)], srcv)
        pltpu.sync_copy(dst_hbm.at[pl.ds(off, _K)], dstv)
        for j in range(_K // 16):
            sl = pl.ds(j * 16, 16)
            srcv[sl] = srcv[sl] + off16
            dstv[sl] = jnp.where(keep16, dstv[sl], trash16)
        pltpu.async_copy(g_hbm.at[srcv], rowsv, sem).wait()
        pltpu.sync_copy(rowsv, acc.at[dstv], add=True)
        return carry

    lax.fori_loop(0, per_tile // _K, body, 0)
    plsc.subcore_barrier()
    for t in range(_RPT // _K):
        r = row0 + t * _K
        pltpu.sync_copy(acc.at[pl.ds(r, _K)], stage)
        pltpu.sync_copy(stage, out_hbm.at[pl.ds(c * _NP + r, _K)])


def _tc1_body(x_ref, w_ref, deg_ref, gs_ref, gf_ref, dinv_ref):
    deg = deg_ref[0, :, 0:1] + deg_ref[1, :, 0:1] + 1.0
    dinv = lax.rsqrt(deg)
    h = jnp.dot(x_ref[...], w_ref[...], preferred_element_type=f32)
    g = h * dinv
    half = _HID // 2
    gs_ref[0] = g[:, :half]
    gs_ref[1] = g[:, half:]
    gf_ref[...] = g
    dinv_ref[...] = dinv


def _tc1(x_pad, W1, degacc):
    grid = (_NP // _RB,)
    return pl.pallas_call(
        _tc1_body,
        grid=grid,
        in_specs=[
            pl.BlockSpec((_RB, _IN), lambda i: (i, 0)),
            pl.BlockSpec((_IN, _HID), lambda i: (0, 0)),
            pl.BlockSpec((2, _RB, 128), lambda i: (0, i, 0)),
        ],
        out_specs=[
            pl.BlockSpec((2, _RB, _HID // 2), lambda i: (0, i, 0)),
            pl.BlockSpec((_RB, _HID), lambda i: (i, 0)),
            pl.BlockSpec((_RB, 1), lambda i: (i, 0)),
        ],
        out_shape=[
            jax.ShapeDtypeStruct((2, _NP, _HID // 2), f32),
            jax.ShapeDtypeStruct((_NP, _HID), f32),
            jax.ShapeDtypeStruct((_NP, 1), f32),
        ],
    )(x_pad, W1, degacc)


def _tc2_body(acc_ref, gf_ref, dinv_ref, b_ref, w_ref, g2f_ref):
    accf = jnp.concatenate([acc_ref[0], acc_ref[1]], axis=1)
    dinv = dinv_ref[...]
    h2 = jnp.maximum(dinv * (accf + gf_ref[...]) + b_ref[...], 0.0)
    g2f_ref[...] = jnp.dot(h2, w_ref[...], preferred_element_type=f32) * dinv


def _tc2(acc1, g1f, dinv, b1, W2):
    grid = (_NP // _RB,)
    return pl.pallas_call(
        _tc2_body,
        grid=grid,
        in_specs=[
            pl.BlockSpec((2, _RB, _HID // 2), lambda i: (0, i, 0)),
            pl.BlockSpec((_RB, _HID), lambda i: (i, 0)),
            pl.BlockSpec((_RB, 1), lambda i: (i, 0)),
            pl.BlockSpec((1, _HID), lambda i: (0, 0)),
            pl.BlockSpec((_HID, _OUT), lambda i: (0, 0)),
        ],
        out_specs=pl.BlockSpec((_RB, _OUT), lambda i: (i, 0)),
        out_shape=jax.ShapeDtypeStruct((_NSC * _NP, _OUT), f32),
    )(acc1, g1f, dinv, b1, W2)


def _tc3_body(acc_ref, g2f_ref, dinv_ref, b_ref, out_ref):
    accf = acc_ref[0] + acc_ref[1]
    out_ref[...] = dinv_ref[...] * (accf + g2f_ref[...]) + b_ref[...]


def _tc3(acc2, g2f, dinv, b2):
    grid = (_NP // _RB,)
    return pl.pallas_call(
        _tc3_body,
        grid=grid,
        in_specs=[
            pl.BlockSpec((2, _RB, _OUT), lambda i: (0, i, 0)),
            pl.BlockSpec((_RB, _OUT), lambda i: (i, 0)),
            pl.BlockSpec((_RB, 1), lambda i: (i, 0)),
            pl.BlockSpec((1, _OUT), lambda i: (0, 0)),
        ],
        out_specs=pl.BlockSpec((_RB, _OUT), lambda i: (i, 0)),
        out_shape=jax.ShapeDtypeStruct((_NP, _OUT), f32),
    )(acc2, g2f, dinv, b2)


@jax.jit
def _run(x, edge_index, W1, b1, W2, b2):
    ei = edge_index.astype(i32)
    pad = jnp.full((_EP - _E,), _N, dtype=i32)
    src = jnp.concatenate([ei[0], pad])
    dst = jnp.concatenate([ei[1], pad])

    x_pad = jnp.zeros((_NP, _IN), f32).at[:_N].set(x)

    zeros_acc = jnp.zeros((_RPT, _F), f32)
    keep_all = jnp.full((16,), 3, i32)
    keep_eo = jnp.where(jnp.arange(16) % 2 == 0, 1, 2).astype(i32)
    off_np = jnp.full((16,), _NP, i32)
    off_0 = jnp.zeros((16,), i32)

    onehot_tbl = jnp.zeros((_NSC * _NP, _F), f32).at[:, 0].set(1.0)
    degacc = _sc_scatter(onehot_tbl, src, dst, zeros_acc, off_0, keep_eo)
    degacc = degacc.reshape(_NSC, _NP, _F)

    g1s, g1f, dinv = _tc1(x_pad, W1, degacc)

    acc1 = _sc_scatter(g1s.reshape(_NSC * _NP, _F), src, dst, zeros_acc,
                       off_np, keep_all)
    acc1 = acc1.reshape(_NSC, _NP, _F)

    g2pad = _tc2(acc1, g1f, dinv, b1.reshape(1, _HID), W2)

    acc2 = _sc_scatter(g2pad, src, dst, zeros_acc, off_0, keep_eo)
    acc2 = acc2.reshape(_NSC, _NP, _OUT)

    out = _tc3(acc2, g2pad, dinv, b2.reshape(1, _OUT))
    return out[:_N]


def kernel(x, edge_index, W1, b1, W2, b2):
    return _run(x, edge_index, W1, b1, W2, b2)
